# baseline (speedup 1.0000x reference)
import functools

import jax
import jax.numpy as jnp
from jax import lax
from jax.experimental import pallas as pl
from jax.experimental.pallas import tpu as pltpu
from jax.experimental.pallas import tpu_sc as plsc

_B = 16384
_NCTX = 5
_D = 32
_LANES = 16

_info = plsc.get_sparse_core_info()
_NC, _NS = _info.num_cores, _info.num_subcores
_NW = _NC * _NS
_BPW = _B // _NW
_CPW = _BPW * _NCTX
_CHUNK = 128

_mesh = plsc.VectorSubcoreMesh(core_axis_name="c", subcore_axis_name="s")


@functools.partial(
    pl.kernel,
    mesh=_mesh,
    out_type=jax.ShapeDtypeStruct((_B * _NCTX,), jnp.float32),
    scratch_types=[
        pltpu.VMEM((_BPW,), jnp.int32),
        pltpu.VMEM((_CPW,), jnp.int32),
        pltpu.VMEM((_BPW, _D), jnp.float32),
        pltpu.VMEM((_CPW, _D), jnp.float32),
        pltpu.VMEM((_CPW,), jnp.float32),
        pltpu.SemaphoreType.DMA,
    ],
    compiler_params=pltpu.CompilerParams(
        needs_layout_passes=False, use_tc_tiling_on_sc=False),
)
def _act2vec_sc(t_hbm, c_hbm, wt_hbm, wc_hbm, out_hbm,
                tix_v, cix_v, we_v, ce_v, out_v, sem):
    wid = lax.axis_index("s") * _NC + lax.axis_index("c")
    tb = wid * _BPW
    cb = wid * _CPW

    pltpu.sync_copy(t_hbm.at[pl.ds(tb, _BPW)], tix_v)
    pltpu.sync_copy(c_hbm.at[pl.ds(cb, _CPW)], cix_v)

    copies = []
    for j in range(_BPW // _CHUNK):
        copies.append(pltpu.async_copy(
            wt_hbm.at[tix_v.at[pl.ds(j * _CHUNK, _CHUNK)]],
            we_v.at[pl.ds(j * _CHUNK, _CHUNK)], sem))
    for j in range(_CPW // _CHUNK):
        copies.append(pltpu.async_copy(
            wc_hbm.at[cix_v.at[pl.ds(j * _CHUNK, _CHUNK)]],
            ce_v.at[pl.ds(j * _CHUNK, _CHUNK)], sem))
    for c in copies:
        c.wait()

    last = lax.iota(jnp.int32, _LANES) == (_LANES - 1)

    def body(b, carry):
        we0 = we_v[b, pl.ds(0, _LANES)]
        we1 = we_v[b, pl.ds(_LANES, _LANES)]
        for n in range(_NCTX):
            r = b * _NCTX + n
            ce0 = ce_v[r, pl.ds(0, _LANES)]
            ce1 = ce_v[r, pl.ds(_LANES, _LANES)]
            p = ce0 * we0 + ce1 * we1
            s = jnp.full((_LANES,), jnp.sum(p))
            plsc.store_scatter(out_v, [jnp.full((_LANES,), r, jnp.int32)],
                               s, mask=last)
        return carry

    lax.fori_loop(0, _BPW, body, 0)

    pltpu.sync_copy(out_v, out_hbm.at[pl.ds(cb, _CPW)])


def kernel(target, context, W_target, W_context):
    tflat = target.reshape(-1)
    cflat = context.reshape(-1)
    out = _act2vec_sc(tflat, cflat, W_target, W_context)
    return out.reshape(_B, _NCTX)
